# Initial kernel scaffold; baseline (speedup 1.0000x reference)
import jax
import jax.numpy as jnp
from jax.experimental import pallas as pl


def kernel(heading_class_label, heading_scores, heading_residual_label, heading_residuals_normalized, object_assignment, objectness_label):
    raise NotImplementedError("write your pallas kernel here")


# speedup vs baseline: 13.9436x; 13.9436x over previous
import functools

import jax
import jax.numpy as jnp
from jax import lax
from jax.experimental import pallas as pl
from jax.experimental.pallas import tpu as pltpu
from jax.experimental.pallas import tpu_sc as plsc

NB = 12
B = 64
K = 1024
G = 128
NC = 2
NS = 16
NW = NC * NS
BPW = B // NW
L = 16
GROUPS = K // L

_LN2 = 0.6931471805599453
_SQRT2 = 1.4142135623730951
_INV_DELTA = float(NB) / 3.141592653589793


def _log_f32(x):
    xi = plsc.bitcast(x, jnp.int32)
    e = (xi >> 23) - 127
    m = plsc.bitcast((xi & 0x007FFFFF) | 0x3F800000, jnp.float32)
    big = m > _SQRT2
    m = jnp.where(big, m * 0.5, m)
    e = jnp.where(big, e + 1, e)
    f = m - 1.0
    s = f / (2.0 + f)
    s2 = s * s
    ln_m = 2.0 * s * (1.0 + s2 * (1.0 / 3.0 + s2 * (0.2 + s2 * (1.0 / 7.0))))
    return e.astype(jnp.float32) * _LN2 + ln_m


def _sc_body(scores_hbm, resid_hbm, cls_hbm, rlab_hbm, oa_hbm, obj_hbm,
             out_hbm, scores_v, resid_v, cls_v, rlab_v, oa_v, obj_v, stage_v):
    wid = lax.axis_index("s") * NC + lax.axis_index("c")
    iota12 = lax.iota(jnp.int32, L) * NB

    acc = (jnp.zeros((L,), jnp.float32),
           jnp.zeros((L,), jnp.float32),
           jnp.zeros((L,), jnp.float32))

    for b_local in range(BPW):
        b = wid * BPW + b_local
        pltpu.sync_copy(scores_hbm.at[b], scores_v)
        pltpu.sync_copy(resid_hbm.at[b], resid_v)
        pltpu.sync_copy(cls_hbm.at[b], cls_v)
        pltpu.sync_copy(rlab_hbm.at[b], rlab_v)
        pltpu.sync_copy(oa_hbm.at[b], oa_v)
        pltpu.sync_copy(obj_hbm.at[b], obj_v)

        def group_body(g, carry):
            acc_ce, acc_hu, acc_obj = carry
            base = g * L
            oa = oa_v[pl.ds(base, L)]
            obj = obj_v[pl.ds(base, L)].astype(jnp.float32)
            hcl = plsc.load_gather(cls_v, [oa])
            idx0 = iota12 + g * (L * NB)
            svals = [plsc.load_gather(scores_v, [idx0 + j]) for j in range(NB)]
            m = svals[0]
            for j in range(1, NB):
                m = jnp.maximum(m, svals[j])
            se = jnp.exp(svals[0] - m)
            for j in range(1, NB):
                se = se + jnp.exp(svals[j] - m)
            lse = _log_f32(se) + m
            s_h = plsc.load_gather(scores_v, [idx0 + hcl])
            ce = lse - s_h
            hrl = plsc.load_gather(rlab_v, [oa]) * _INV_DELTA
            rn = plsc.load_gather(resid_v, [idx0 + hcl])
            err = rn - hrl
            ae = jnp.abs(err)
            q = jnp.minimum(ae, 1.0)
            hub = 0.5 * q * q + (ae - q)
            return (acc_ce + ce * obj, acc_hu + hub * obj, acc_obj + obj)

        acc = lax.fori_loop(0, GROUPS, group_body, acc)

    stage_v[pl.ds(0, L)] = acc[0]
    stage_v[pl.ds(L, L)] = acc[1]
    stage_v[pl.ds(2 * L, L)] = acc[2]
    pltpu.sync_copy(stage_v, out_hbm.at[wid])


@jax.jit
def kernel(heading_class_label, heading_scores, heading_residual_label,
           heading_residuals_normalized, object_assignment, objectness_label):
    scores2 = heading_scores.reshape(B, K * NB)
    resid2 = heading_residuals_normalized.reshape(B, K * NB)
    cls2 = heading_class_label.astype(jnp.int32)
    oa2 = object_assignment.astype(jnp.int32)
    obj2 = objectness_label.astype(jnp.int32)

    mesh = plsc.VectorSubcoreMesh(core_axis_name="c", subcore_axis_name="s",
                                  num_cores=NC, num_subcores=NS)
    partials = pl.kernel(
        _sc_body,
        out_type=jax.ShapeDtypeStruct((NW, 3 * L), jnp.float32),
        mesh=mesh,
        compiler_params=pltpu.CompilerParams(needs_layout_passes=False),
        scratch_types=[
            pltpu.VMEM((K * NB,), jnp.float32),
            pltpu.VMEM((K * NB,), jnp.float32),
            pltpu.VMEM((G,), jnp.int32),
            pltpu.VMEM((G,), jnp.float32),
            pltpu.VMEM((K,), jnp.int32),
            pltpu.VMEM((K,), jnp.int32),
            pltpu.VMEM((3 * L,), jnp.float32),
        ],
    )(scores2, resid2, cls2, heading_residual_label, oa2, obj2)

    sums = partials.reshape(NW, 3, L).sum(axis=(0, 2))
    denom = sums[2] + 1e-6
    return (sums[0] / denom, sums[1] / denom)
